# baseline (speedup 1.0000x reference)
import jax
import jax.numpy as jnp
from jax import lax
from jax.experimental import pallas as pl
from jax.experimental.pallas import tpu as pltpu
from jax.experimental.pallas import tpu_sc as plsc

VOCAB = 1000
VOCAB2 = VOCAB // 2
BATCH = 1024
HIST = 50
NC, NS = 2, 16
NW = NC * NS
BPW = BATCH // NW
IDS_PAD = 64
ONE_LO = 0x3F80
ONE_HI = 0x3F800000

GROUPS = [(0, False), (16, False), (32, False), (40, True)]


def _sc_body(ids_ref, zeros_ref, out_ref, buf, ids_v, stash, sems):
    wid = lax.axis_index("s") * NC + lax.axis_index("c")
    base = wid * BPW
    iota = lax.iota(jnp.int32, 16)
    zeros16 = jnp.zeros((16,), jnp.int32)

    pltpu.sync_copy(zeros_ref, buf.at[0])
    pltpu.sync_copy(zeros_ref, buf.at[1])

    for t in range(BPW):
        s = t % 2
        if t >= 2:
            pltpu.make_async_copy(
                buf.at[s], out_ref.at[base + (t - 2)], sems.at[s]).wait()
            for j, (off, need_mask) in enumerate(GROUPS):
                rows = iota + off
                prev = stash[s, j, :]
                if need_mask:
                    plsc.store_scatter(buf.at[s], [rows, prev], zeros16,
                                       mask=rows < HIST)
                else:
                    plsc.store_scatter(buf.at[s], [rows, prev], zeros16)
        pltpu.sync_copy(ids_ref.at[base + t], ids_v)
        for j, (off, need_mask) in enumerate(GROUPS):
            rows = iota + off
            idx = ids_v[pl.ds(off, 16)]
            col = lax.shift_right_logical(idx, 1)
            val = jnp.where((idx & 1) == 0,
                            jnp.full((16,), ONE_LO, jnp.int32),
                            jnp.full((16,), ONE_HI, jnp.int32))
            stash[s, j, :] = col
            if need_mask:
                plsc.store_scatter(buf.at[s], [rows, col], val,
                                   mask=rows < HIST)
            else:
                plsc.store_scatter(buf.at[s], [rows, col], val)
        pltpu.make_async_copy(
            buf.at[s], out_ref.at[base + t], sems.at[s]).start()

    for t in (BPW - 2, BPW - 1):
        s = t % 2
        pltpu.make_async_copy(
            buf.at[s], out_ref.at[base + t], sems.at[s]).wait()


def kernel(input):
    ids = input.astype(jnp.int32)
    ids_p = jnp.pad(ids, ((0, 0), (0, IDS_PAD - HIST)))
    zeros = jnp.zeros((HIST, VOCAB2), jnp.int32)
    mesh = plsc.VectorSubcoreMesh(core_axis_name="c", subcore_axis_name="s")
    f = pl.kernel(
        _sc_body,
        mesh=mesh,
        out_type=jax.ShapeDtypeStruct((BATCH, HIST, VOCAB2), jnp.int32),
        scratch_types=[
            pltpu.VMEM((2, HIST, VOCAB2), jnp.int32),
            pltpu.VMEM((IDS_PAD,), jnp.int32),
            pltpu.VMEM((2, len(GROUPS), 16), jnp.int32),
            pltpu.SemaphoreType.DMA((2,)),
        ],
        compiler_params=pltpu.CompilerParams(needs_layout_passes=False),
    )
    packed = f(ids_p, zeros)
    bf = jax.lax.bitcast_convert_type(packed, jnp.bfloat16)
    return bf.reshape(BATCH, HIST, VOCAB).astype(jnp.float32)


# speedup vs baseline: 2.1241x; 2.1241x over previous
import jax
import jax.numpy as jnp
from jax import lax
from jax.experimental import pallas as pl
from jax.experimental.pallas import tpu as pltpu
from jax.experimental.pallas import tpu_sc as plsc

VOCAB = 1000
BATCH = 1024
HIST = 50
NC, NS = 2, 16
NW = NC * NS
BPW = BATCH // NW
IDS_PAD = 64

GROUPS = [(0, False), (16, False), (32, False), (40, True)]


def _sc_body(ids_ref, zeros_ref, out_ref, buf, ids_v, stash, sems):
    wid = lax.axis_index("s") * NC + lax.axis_index("c")
    base = wid * BPW
    iota = lax.iota(jnp.int32, 16)
    ones16 = jnp.full((16,), 1.0, jnp.float32)
    zeros16 = jnp.zeros((16,), jnp.float32)

    pltpu.sync_copy(zeros_ref, buf.at[0])
    pltpu.sync_copy(zeros_ref, buf.at[1])

    for t in range(BPW):
        s = t % 2
        if t >= 2:
            pltpu.make_async_copy(
                buf.at[s], out_ref.at[base + (t - 2)], sems.at[s]).wait()
            for j, (off, need_mask) in enumerate(GROUPS):
                rows = iota + off
                prev = stash[s, j, :]
                if need_mask:
                    plsc.store_scatter(buf.at[s], [rows, prev], zeros16,
                                       mask=rows < HIST)
                else:
                    plsc.store_scatter(buf.at[s], [rows, prev], zeros16)
        pltpu.sync_copy(ids_ref.at[base + t], ids_v)
        for j, (off, need_mask) in enumerate(GROUPS):
            rows = iota + off
            idx = ids_v[pl.ds(off, 16)]
            stash[s, j, :] = idx
            if need_mask:
                plsc.store_scatter(buf.at[s], [rows, idx], ones16,
                                   mask=rows < HIST)
            else:
                plsc.store_scatter(buf.at[s], [rows, idx], ones16)
        pltpu.make_async_copy(
            buf.at[s], out_ref.at[base + t], sems.at[s]).start()

    for t in (BPW - 2, BPW - 1):
        s = t % 2
        pltpu.make_async_copy(
            buf.at[s], out_ref.at[base + t], sems.at[s]).wait()


def kernel(input):
    ids = input.astype(jnp.int32)
    ids_p = jnp.pad(ids, ((0, 0), (0, IDS_PAD - HIST)))
    zeros = jnp.zeros((HIST, VOCAB), jnp.float32)
    mesh = plsc.VectorSubcoreMesh(core_axis_name="c", subcore_axis_name="s")
    f = pl.kernel(
        _sc_body,
        mesh=mesh,
        out_type=jax.ShapeDtypeStruct((BATCH, HIST, VOCAB), jnp.float32),
        scratch_types=[
            pltpu.VMEM((2, HIST, VOCAB), jnp.float32),
            pltpu.VMEM((IDS_PAD,), jnp.int32),
            pltpu.VMEM((2, len(GROUPS), 16), jnp.int32),
            pltpu.SemaphoreType.DMA((2,)),
        ],
        compiler_params=pltpu.CompilerParams(needs_layout_passes=False),
    )
    return f(ids_p, zeros)
